# baseline (device time: 41442 ns/iter reference)
import jax
import jax.numpy as jnp
from jax import lax
from jax.experimental import pallas as pl
from jax.experimental.pallas import tpu as pltpu

N_DEV = 4
B, S, D = 2, 256, 512
R = B * S
C = R // N_DEV
H_LOCAL = 4
DH = 64
EPS = 1e-5
BF = jnp.bfloat16
F32 = jnp.float32


def _ln_mod(xb, scale_row, shift_row):
    m = jnp.mean(xb, axis=-1, keepdims=True)
    v = jnp.mean((xb - m) ** 2, axis=-1, keepdims=True)
    xn = (xb - m) * lax.rsqrt(v + EPS)
    return xn * (1.0 + scale_row) + shift_row


def _row(v, b):
    return jnp.where(b == 0, v[0:1, :], v[1:2, :])


def kernel(x, Wq, Wk, Wv, Wo, t_emb, W_mod, W_ff1, W_ff2):
    def body(x_ref, wq_ref, wk_ref, wv_ref, wo_ref, temb_ref, wmod_ref,
             wff1_ref, wff2_ref, out_ref,
             sbufA, commRS1, sbufAG1, commAG1,
             sbufF, commRS2, sbufAG2, commAG2,
             ssRS1, srRS1, ssAG1, srAG1, ssRS2, srRS2, ssAG2, srAG2):
        my = lax.axis_index("i")

        def send(src, dst, ssem, rsem, dev):
            rdma = pltpu.make_async_remote_copy(
                src_ref=src, dst_ref=dst, send_sem=ssem, recv_sem=rsem,
                device_id=(dev,), device_id_type=pl.DeviceIdType.MESH)
            rdma.start()
            return rdma

        def wait_recv(dst, rsem):
            rdma = pltpu.make_async_remote_copy(
                src_ref=dst, dst_ref=dst, send_sem=rsem, recv_sem=rsem,
                device_id=(0,), device_id_type=pl.DeviceIdType.MESH)
            rdma.wait_recv()

        mod = jnp.dot(temb_ref[:].astype(BF), wmod_ref[:].astype(BF),
                      preferred_element_type=F32)
        sa, sha, ga, sm, shm, gm = [mod[:, i * D:(i + 1) * D]
                                    for i in range(6)]

        wq = wq_ref[:].astype(BF)
        wk = wk_ref[:].astype(BF)
        wv = wv_ref[:].astype(BF)
        wo = wo_ref[:].astype(BF)

        rs1_sends = []
        chunk_partials = []
        for b in range(B):
            x0b = x_ref[b * S:(b + 1) * S, :]
            xb = _ln_mod(x0b, sa[b:b + 1, :], sha[b:b + 1, :]).astype(BF)
            Q = jnp.dot(xb, wq, preferred_element_type=F32)
            K = jnp.dot(xb, wk, preferred_element_type=F32)
            V = jnp.dot(xb, wv, preferred_element_type=F32)
            for half in range(2):
                c = 2 * b + half
                r0 = half * C
                heads = []
                for h in range(H_LOCAL):
                    q = Q[r0:r0 + C, h * DH:(h + 1) * DH].astype(BF)
                    kk = K[:, h * DH:(h + 1) * DH].astype(BF)
                    v = V[:, h * DH:(h + 1) * DH].astype(BF)
                    s = lax.dot_general(q, kk, (((1,), (1,)), ((), ())),
                                        preferred_element_type=F32) * 0.125
                    s = s - jnp.max(s, axis=-1, keepdims=True)
                    p = jnp.exp(s)
                    p = p / jnp.sum(p, axis=-1, keepdims=True)
                    heads.append(jnp.dot(p.astype(BF), v,
                                         preferred_element_type=F32).astype(BF))
                ob = jnp.concatenate(heads, axis=1)
                pc = jnp.dot(ob, wo, preferred_element_type=F32)
                chunk_partials.append(pc)
                sbufA[c, :, :] = pc.astype(BF)
                k_rel = (c - my) % N_DEV

                @pl.when(c != my)
                def _(c=c, k_rel=k_rel):
                    rs1_sends.append(send(
                        sbufA.at[c], commRS1.at[k_rel - 1],
                        ssRS1.at[c], srRS1.at[k_rel - 1], c))

        for j in range(N_DEV - 1):
            wait_recv(commRS1.at[j], srRS1.at[j])
        own = jnp.where(
            my == 0, chunk_partials[0],
            jnp.where(my == 1, chunk_partials[1],
                      jnp.where(my == 2, chunk_partials[2],
                                chunk_partials[3])))
        red = own
        for j in range(N_DEV - 1):
            red = red + commRS1[j].astype(F32)

        sbufAG1[:, :] = red.astype(BF)
        ag1_sends = []
        for k in range(1, N_DEV):
            ag1_sends.append(send(
                sbufAG1, commAG1.at[k - 1],
                ssAG1.at[k - 1], srAG1.at[k - 1], (my + k) % N_DEV))

        wff1 = wff1_ref[:].astype(BF)
        wff2 = wff2_ref[:].astype(BF)

        def ff_chunk(c, attn_sum_c):
            x0c = x_ref[pl.ds(c * C, C), :]
            bb = c // 2
            x1c = x0c + _row(ga, bb) * attn_sum_c
            xm = _ln_mod(x1c, _row(sm, bb), _row(shm, bb)).astype(BF)
            h = jnp.dot(xm, wff1, preferred_element_type=F32)
            h = h * (1.0 / (1.0 + jnp.exp(-h)))
            p2 = jnp.dot(h.astype(BF), wff2, preferred_element_type=F32)
            return x1c, p2

        x1_vals = []
        x1_my, ffp_own = ff_chunk(my, red)
        x1_vals.append(x1_my)
        rs2_sends = []
        for t in range(1, N_DEV):
            c = (my + N_DEV - t) % N_DEV
            wait_recv(commAG1.at[t - 1], srAG1.at[t - 1])
            x1c, p2 = ff_chunk(c, commAG1[t - 1].astype(F32))
            x1_vals.append(x1c)
            sbufF[t - 1, :, :] = p2.astype(BF)
            rs2_sends.append(send(
                sbufF.at[t - 1], commRS2.at[3 - t],
                ssRS2.at[t - 1], srRS2.at[3 - t], c))

        for j in range(N_DEV - 1):
            wait_recv(commRS2.at[j], srRS2.at[j])
        ffsum = ffp_own
        for j in range(N_DEV - 1):
            ffsum = ffsum + commRS2[j].astype(F32)
        out_my = x1_vals[0] + _row(gm, my // 2) * ffsum
        out_ref[pl.ds(my * C, C), :] = out_my

        sbufAG2[:, :] = ffsum.astype(BF)
        ag2_sends = []
        for k in range(1, N_DEV):
            ag2_sends.append(send(
                sbufAG2, commAG2.at[k - 1],
                ssAG2.at[k - 1], srAG2.at[k - 1], (my + k) % N_DEV))
        for t in range(1, N_DEV):
            c = (my + N_DEV - t) % N_DEV
            wait_recv(commAG2.at[t - 1], srAG2.at[t - 1])
            out_c = x1_vals[t] + _row(gm, c // 2) * commAG2[t - 1].astype(F32)
            out_ref[pl.ds(c * C, C), :] = out_c

        for c in range(N_DEV):
            @pl.when(c != my)
            def _(c=c):
                pltpu.make_async_remote_copy(
                    src_ref=sbufA.at[c], dst_ref=sbufA.at[c],
                    send_sem=ssRS1.at[c], recv_sem=ssRS1.at[c],
                    device_id=(0,), device_id_type=pl.DeviceIdType.MESH,
                ).wait_send()
        for lst in (ag1_sends, rs2_sends, ag2_sends):
            for rdma in lst:
                rdma.wait_send()

    out2d = pl.pallas_call(
        body,
        out_shape=jax.ShapeDtypeStruct((R, D), F32),
        in_specs=[pl.BlockSpec(memory_space=pltpu.VMEM)] * 9,
        out_specs=pl.BlockSpec(memory_space=pltpu.VMEM),
        scratch_shapes=[
            pltpu.VMEM((N_DEV, C, D), BF),
            pltpu.VMEM((N_DEV - 1, C, D), BF),
            pltpu.VMEM((C, D), BF),
            pltpu.VMEM((N_DEV - 1, C, D), BF),
            pltpu.VMEM((N_DEV - 1, C, D), BF),
            pltpu.VMEM((N_DEV - 1, C, D), BF),
            pltpu.VMEM((C, D), BF),
            pltpu.VMEM((N_DEV - 1, C, D), BF),
            pltpu.SemaphoreType.DMA((N_DEV,)),
            pltpu.SemaphoreType.DMA((N_DEV - 1,)),
            pltpu.SemaphoreType.DMA((N_DEV - 1,)),
            pltpu.SemaphoreType.DMA((N_DEV - 1,)),
            pltpu.SemaphoreType.DMA((N_DEV - 1,)),
            pltpu.SemaphoreType.DMA((N_DEV - 1,)),
            pltpu.SemaphoreType.DMA((N_DEV - 1,)),
            pltpu.SemaphoreType.DMA((N_DEV - 1,)),
        ],
    )(x.reshape(R, D), Wq, Wk, Wv, Wo, t_emb, W_mod, W_ff1, W_ff2)
    return out2d.reshape(B, S, D)
